# baseline (speedup 1.0000x reference)
import jax
import jax.numpy as jnp
from jax.experimental import pallas as pl

_K = 8192
_D = 256
_TILE_M = 256
_CHUNKS = ((0, 2736), (2736, 5472), (5472, _K))


def _vq_kernel(x_ref, w_ref, zsq_ref, wsq_ref, out_ref):
    x = x_ref[...]
    w = w_ref[...]
    cross = jax.lax.dot_general(
        x, w, (((1,), (1,)), ((), ())),
        preferred_element_type=jnp.float32)
    dists = (zsq_ref[...] + wsq_ref[...]) - 2.0 * cross
    iota = jax.lax.broadcasted_iota(jnp.int32, dists.shape, 1)

    v = jnp.full((_TILE_M, 1), jnp.inf, jnp.float32)
    idx = jnp.zeros((_TILE_M, 1), jnp.int32)
    for lo, hi in _CHUNKS:
        mask = (iota >= lo) & (iota < hi)
        dd = jnp.where(mask, dists, jnp.inf)
        m = jnp.min(dd, axis=1, keepdims=True)
        ii = jnp.min(jnp.where(dd == m, iota, jnp.int32(_K)),
                     axis=1, keepdims=True)
        vb = v.astype(jnp.bfloat16).astype(jnp.float32)
        take = m < vb
        v = jnp.where(take, m, vb)
        idx = jnp.where(take, ii, idx)
    out_ref[...] = idx.reshape(1, 1, _TILE_M)


def kernel(z_e_x, embedding_weight):
    b, t, d = z_e_x.shape
    n = b * t
    flat = z_e_x.reshape(n, d)
    zsq = jnp.sum(flat * flat, axis=1, keepdims=True)
    wsq = jnp.sum(embedding_weight * embedding_weight, axis=1)[None]
    grid = n // _TILE_M
    out = pl.pallas_call(
        _vq_kernel,
        grid=(grid,),
        in_specs=[
            pl.BlockSpec((_TILE_M, d), lambda i: (i, 0)),
            pl.BlockSpec((_K, d), lambda i: (0, 0)),
            pl.BlockSpec((_TILE_M, 1), lambda i: (i, 0)),
            pl.BlockSpec((1, _K), lambda i: (0, 0)),
        ],
        out_specs=pl.BlockSpec((1, 1, _TILE_M), lambda i: (i, 0, 0)),
        out_shape=jax.ShapeDtypeStruct((grid, 1, _TILE_M), jnp.int32),
    )(flat, embedding_weight, zsq, wsq)
    return out.reshape(b, t)
